# baseline (speedup 1.0000x reference)
import functools

import jax
import jax.numpy as jnp
from jax.experimental import pallas as pl
from jax.experimental.pallas import tpu as pltpu

BN_EPS = 1e-5


def _fused_kernel(x_ref, w_ref, wbf_ref, gb_ref, o_ref,
                  sx_acc, g_acc, aff_acc, *, n_stats, inv_m):
    t = pl.program_id(1)

    @pl.when(t == 0)
    def _init():
        sx_acc[...] = jnp.zeros_like(sx_acc)
        g_acc[...] = jnp.zeros_like(g_acc)

    @pl.when(t < n_stats)
    def _stats():
        x = x_ref[...]
        xf = x.astype(jnp.float32)
        sx_acc[...] += jnp.sum(xf, axis=(0, 2))[None, :]
        g_b = jax.lax.dot_general(
            x, x, (((2,), (2,)), ((0,), (0,))),
            preferred_element_type=jnp.float32)
        g_acc[...] += jnp.sum(g_b, axis=0)

    @pl.when(t == n_stats - 1)
    def _fold():
        wf = w_ref[...]
        mean_y = jnp.sum(wf * (sx_acc[...] * inv_m), axis=1, keepdims=True)
        wg = jnp.dot(wf, g_acc[...], preferred_element_type=jnp.float32)
        e_y2 = jnp.sum(wg * wf, axis=1, keepdims=True) * inv_m
        var = e_y2 - mean_y * mean_y
        inv_std = jax.lax.rsqrt(var + BN_EPS)
        a = gb_ref[:, 0:1] * inv_std
        aff_acc[:, 0:1] = a
        aff_acc[:, 1:2] = gb_ref[:, 1:2] - mean_y * a

    @pl.when(t >= n_stats)
    def _apply():
        w = wbf_ref[...]
        scale = aff_acc[:, 0:1]
        shift = aff_acc[:, 1:2]
        for b in range(x_ref.shape[0]):
            y = jnp.dot(w, x_ref[b], preferred_element_type=jnp.float32)
            o_ref[b] = y * scale + shift


def kernel(x_nchw, conv_weight, gamma, beta):
    stride = 2
    N, Cin, H, W = x_nchw.shape
    Cout = conv_weight.shape[0]

    xs = x_nchw[:, :, ::stride, ::stride]
    Ho, Wo = xs.shape[2], xs.shape[3]
    Hs = Ho * Wo
    x3 = xs.reshape(N, Cin, Hs).astype(jnp.bfloat16)

    w = conv_weight[:, :, 0, 0]
    w_bf = w.astype(jnp.bfloat16)
    gb = jnp.stack([gamma.astype(jnp.float32),
                    beta.astype(jnp.float32)], axis=1)

    BT = 8
    while N % BT or (N // BT) % 2:
        BT -= 1
    T = N // BT
    TA = T // 2
    inv_m = 1.0 / float(N * Hs)

    x_map = lambda c, t: (jnp.where(t < T, t, c * TA + t - T), 0, 0)
    o_map = lambda c, t: (c * TA + jnp.where(t < T, 0, t - T), 0, 0)

    out3 = pl.pallas_call(
        functools.partial(_fused_kernel, n_stats=T, inv_m=inv_m),
        out_shape=jax.ShapeDtypeStruct((N, Cout, Hs), x_nchw.dtype),
        grid=(2, T + TA),
        in_specs=[pl.BlockSpec((BT, Cin, Hs), x_map),
                  pl.BlockSpec((Cout, Cin), lambda c, t: (0, 0)),
                  pl.BlockSpec((Cout, Cin), lambda c, t: (0, 0)),
                  pl.BlockSpec((Cout, 2), lambda c, t: (0, 0))],
        out_specs=pl.BlockSpec((BT, Cout, Hs), o_map),
        scratch_shapes=[pltpu.VMEM((1, Cin), jnp.float32),
                        pltpu.VMEM((Cin, Cin), jnp.float32),
                        pltpu.VMEM((Cout, 2), jnp.float32)],
        compiler_params=pltpu.CompilerParams(
            dimension_semantics=("parallel", "arbitrary")),
    )(x3, w, w_bf, gb)

    return out3.reshape(N, Cout, Ho, Wo)


# speedup vs baseline: 1.2298x; 1.2298x over previous
import functools

import jax
import jax.numpy as jnp
from jax.experimental import pallas as pl
from jax.experimental.pallas import tpu as pltpu

BN_EPS = 1e-5


def _fused_kernel(x_ref, w_ref, wbf_ref, gb_ref, o_ref,
                  y_acc, sx_acc, g_acc, aff_acc, *,
                  n_stats, bt, bt_out, inv_m):
    t = pl.program_id(0)

    @pl.when(t == 0)
    def _init():
        sx_acc[...] = jnp.zeros_like(sx_acc)
        g_acc[...] = jnp.zeros_like(g_acc)

    @pl.when(t < n_stats)
    def _stats_conv():
        xf = x_ref[...]
        x = xf.astype(jnp.bfloat16)
        sx_acc[...] += jnp.sum(xf, axis=(0, 2))[None, :]
        g_b = jax.lax.dot_general(
            x, x, (((2,), (2,)), ((0,), (0,))),
            preferred_element_type=jnp.float32)
        g_acc[...] += jnp.sum(g_b, axis=0)
        w = wbf_ref[...]
        for b in range(bt):
            y_acc[t * bt + b] = jnp.dot(w, x[b],
                                        preferred_element_type=jnp.float32)

    @pl.when(t == n_stats - 1)
    def _fold():
        wf = w_ref[...]
        mean_y = jnp.sum(wf * (sx_acc[...] * inv_m), axis=1, keepdims=True)
        wg = jnp.dot(wf, g_acc[...], preferred_element_type=jnp.float32)
        e_y2 = jnp.sum(wg * wf, axis=1, keepdims=True) * inv_m
        var = e_y2 - mean_y * mean_y
        inv_std = jax.lax.rsqrt(var + BN_EPS)
        a = gb_ref[:, 0:1] * inv_std
        aff_acc[:, 0:1] = a
        aff_acc[:, 1:2] = gb_ref[:, 1:2] - mean_y * a

    @pl.when(t >= n_stats)
    def _apply():
        j = t - n_stats
        scale = aff_acc[:, 0:1][None]
        shift = aff_acc[:, 1:2][None]
        o_ref[...] = y_acc[pl.ds(j * bt_out, bt_out)] * scale + shift


def kernel(x_nchw, conv_weight, gamma, beta):
    stride = 2
    N, Cin, H, W = x_nchw.shape
    Cout = conv_weight.shape[0]

    xs = x_nchw[:, :, ::stride, ::stride]
    Ho, Wo = xs.shape[2], xs.shape[3]
    Hs = Ho * Wo
    x3 = xs.reshape(N, Cin, Hs)

    w = conv_weight[:, :, 0, 0]
    w_bf = w.astype(jnp.bfloat16)
    gb = jnp.stack([gamma.astype(jnp.float32),
                    beta.astype(jnp.float32)], axis=1)

    BT = 4
    while N % BT:
        BT -= 1
    T = N // BT
    BT_OUT = 8
    while N % BT_OUT:
        BT_OUT -= 1
    TA = N // BT_OUT
    inv_m = 1.0 / float(N * Hs)

    x_map = lambda t: (jnp.minimum(t, T - 1), 0, 0)
    o_map = lambda t: (jnp.maximum(t - T, 0), 0, 0)

    out3 = pl.pallas_call(
        functools.partial(_fused_kernel, n_stats=T, bt=BT,
                          bt_out=BT_OUT, inv_m=inv_m),
        out_shape=jax.ShapeDtypeStruct((N, Cout, Hs), x_nchw.dtype),
        grid=(T + TA,),
        in_specs=[pl.BlockSpec((BT, Cin, Hs), x_map),
                  pl.BlockSpec((Cout, Cin), lambda t: (0, 0)),
                  pl.BlockSpec((Cout, Cin), lambda t: (0, 0)),
                  pl.BlockSpec((Cout, 2), lambda t: (0, 0))],
        out_specs=pl.BlockSpec((BT_OUT, Cout, Hs), o_map),
        scratch_shapes=[pltpu.VMEM((N, Cout, Hs), jnp.float32),
                        pltpu.VMEM((1, Cin), jnp.float32),
                        pltpu.VMEM((Cin, Cin), jnp.float32),
                        pltpu.VMEM((Cout, 2), jnp.float32)],
        compiler_params=pltpu.CompilerParams(
            dimension_semantics=("arbitrary",)),
    )(x3, w, w_bf, gb)

    return out3.reshape(N, Cout, Ho, Wo)


# speedup vs baseline: 1.2799x; 1.0407x over previous
import functools

import jax
import jax.numpy as jnp
from jax.experimental import pallas as pl
from jax.experimental.pallas import tpu as pltpu

BN_EPS = 1e-5


def _fused_kernel(x_ref, w_ref, wbf_ref, gb_ref, o_ref,
                  y_acc, sx_acc, g_acc, aff_acc, *,
                  n_stats, bt, bt_out, inv_m):
    t = pl.program_id(0)

    @pl.when(t == 0)
    def _init():
        sx_acc[...] = jnp.zeros_like(sx_acc)
        g_acc[...] = jnp.zeros_like(g_acc)

    @pl.when(t < n_stats)
    def _stats_conv():
        xf = x_ref[...]
        x = xf.astype(jnp.bfloat16)
        sx_acc[...] += jnp.sum(xf, axis=(0, 2))[None, :]
        g_b = jax.lax.dot_general(
            x, x, (((2,), (2,)), ((0,), (0,))),
            preferred_element_type=jnp.float32)
        g_acc[...] += jnp.sum(g_b, axis=0)
        w = wbf_ref[...]
        for b in range(bt):
            y_acc[t * bt + b] = jnp.dot(w, x[b],
                                        preferred_element_type=jnp.float32)

    @pl.when(t == n_stats - 1)
    def _fold():
        wf = w_ref[...]
        mean_y = jnp.sum(wf * (sx_acc[...] * inv_m), axis=1, keepdims=True)
        wg = jnp.dot(wf, g_acc[...], preferred_element_type=jnp.float32)
        e_y2 = jnp.sum(wg * wf, axis=1, keepdims=True) * inv_m
        var = e_y2 - mean_y * mean_y
        inv_std = jax.lax.rsqrt(var + BN_EPS)
        a = gb_ref[:, 0:1] * inv_std
        aff_acc[:, 0:1] = a
        aff_acc[:, 1:2] = gb_ref[:, 1:2] - mean_y * a

    @pl.when(t >= n_stats)
    def _apply():
        j = t - n_stats
        scale = aff_acc[:, 0:1][None]
        shift = aff_acc[:, 1:2][None]
        o_ref[...] = y_acc[pl.ds(j * bt_out, bt_out)] * scale + shift


def kernel(x_nchw, conv_weight, gamma, beta):
    stride = 2
    N, Cin, H, W = x_nchw.shape
    Cout = conv_weight.shape[0]

    xs = x_nchw[:, :, ::stride, ::stride]
    Ho, Wo = xs.shape[2], xs.shape[3]
    Hs = Ho * Wo
    x3 = xs.reshape(N, Cin, Hs)

    w = conv_weight[:, :, 0, 0]
    w_bf = w.astype(jnp.bfloat16)
    gb = jnp.stack([gamma.astype(jnp.float32),
                    beta.astype(jnp.float32)], axis=1)

    BT = 8
    while N % BT:
        BT -= 1
    T = N // BT
    BT_OUT = 8
    while N % BT_OUT:
        BT_OUT -= 1
    TA = N // BT_OUT
    inv_m = 1.0 / float(N * Hs)

    x_map = lambda t: (jnp.minimum(t, T - 1), 0, 0)
    o_map = lambda t: (jnp.maximum(t - T, 0), 0, 0)

    out3 = pl.pallas_call(
        functools.partial(_fused_kernel, n_stats=T, bt=BT,
                          bt_out=BT_OUT, inv_m=inv_m),
        out_shape=jax.ShapeDtypeStruct((N, Cout, Hs), x_nchw.dtype),
        grid=(T + TA,),
        in_specs=[pl.BlockSpec((BT, Cin, Hs), x_map),
                  pl.BlockSpec((Cout, Cin), lambda t: (0, 0)),
                  pl.BlockSpec((Cout, Cin), lambda t: (0, 0)),
                  pl.BlockSpec((Cout, 2), lambda t: (0, 0))],
        out_specs=pl.BlockSpec((BT_OUT, Cout, Hs), o_map),
        scratch_shapes=[pltpu.VMEM((N, Cout, Hs), jnp.float32),
                        pltpu.VMEM((1, Cin), jnp.float32),
                        pltpu.VMEM((Cin, Cin), jnp.float32),
                        pltpu.VMEM((Cout, 2), jnp.float32)],
        compiler_params=pltpu.CompilerParams(
            dimension_semantics=("arbitrary",)),
    )(x3, w, w_bf, gb)

    return out3.reshape(N, Cout, Ho, Wo)


# speedup vs baseline: 1.3104x; 1.0239x over previous
import functools

import jax
import jax.numpy as jnp
from jax.experimental import pallas as pl
from jax.experimental.pallas import tpu as pltpu

BN_EPS = 1e-5


def _fused_kernel(x_ref, w_ref, wbf_ref, gb_ref, o_ref,
                  y_acc, sx_acc, g_acc, aff_acc, *,
                  n_stats, bt, bt_out, inv_m):
    t = pl.program_id(0)

    @pl.when(t == 0)
    def _init():
        sx_acc[...] = jnp.zeros_like(sx_acc)
        g_acc[...] = jnp.zeros_like(g_acc)

    @pl.when(t < n_stats)
    def _stats_conv():
        xf = x_ref[...]
        x = xf.astype(jnp.bfloat16)
        sx_acc[...] += jnp.sum(xf, axis=(0, 2))[None, :]
        g_b = jax.lax.dot_general(
            x, x, (((2,), (2,)), ((0,), (0,))),
            preferred_element_type=jnp.float32)
        g_acc[...] += jnp.sum(g_b, axis=0)
        w = wbf_ref[...]
        for b in range(bt):
            y_acc[t * bt + b] = jnp.dot(w, x[b],
                                        preferred_element_type=jnp.float32)

    @pl.when(t == n_stats - 1)
    def _fold():
        wf = w_ref[...]
        mean_y = jnp.sum(wf * (sx_acc[...] * inv_m), axis=1, keepdims=True)
        wg = jnp.dot(wf, g_acc[...], preferred_element_type=jnp.float32)
        e_y2 = jnp.sum(wg * wf, axis=1, keepdims=True) * inv_m
        var = e_y2 - mean_y * mean_y
        inv_std = jax.lax.rsqrt(var + BN_EPS)
        a = gb_ref[:, 0:1] * inv_std
        aff_acc[:, 0:1] = a
        aff_acc[:, 1:2] = gb_ref[:, 1:2] - mean_y * a

    @pl.when(t >= n_stats)
    def _apply():
        j = t - n_stats
        scale = aff_acc[:, 0:1][None]
        shift = aff_acc[:, 1:2][None]
        o_ref[...] = y_acc[pl.ds(j * bt_out, bt_out)] * scale + shift


def kernel(x_nchw, conv_weight, gamma, beta):
    stride = 2
    N, Cin, H, W = x_nchw.shape
    Cout = conv_weight.shape[0]

    xs = x_nchw[:, :, ::stride, ::stride]
    Ho, Wo = xs.shape[2], xs.shape[3]
    Hs = Ho * Wo
    x3 = xs.reshape(N, Cin, Hs)

    w = conv_weight[:, :, 0, 0]
    w_bf = w.astype(jnp.bfloat16)
    gb = jnp.stack([gamma.astype(jnp.float32),
                    beta.astype(jnp.float32)], axis=1)

    BT = 16
    while N % BT:
        BT -= 1
    T = N // BT
    BT_OUT = 8
    while N % BT_OUT:
        BT_OUT -= 1
    TA = N // BT_OUT
    inv_m = 1.0 / float(N * Hs)

    x_map = lambda t: (jnp.minimum(t, T - 1), 0, 0)
    o_map = lambda t: (jnp.maximum(t - T, 0), 0, 0)

    out3 = pl.pallas_call(
        functools.partial(_fused_kernel, n_stats=T, bt=BT,
                          bt_out=BT_OUT, inv_m=inv_m),
        out_shape=jax.ShapeDtypeStruct((N, Cout, Hs), x_nchw.dtype),
        grid=(T + TA,),
        in_specs=[pl.BlockSpec((BT, Cin, Hs), x_map),
                  pl.BlockSpec((Cout, Cin), lambda t: (0, 0)),
                  pl.BlockSpec((Cout, Cin), lambda t: (0, 0)),
                  pl.BlockSpec((Cout, 2), lambda t: (0, 0))],
        out_specs=pl.BlockSpec((BT_OUT, Cout, Hs), o_map),
        scratch_shapes=[pltpu.VMEM((N, Cout, Hs), jnp.float32),
                        pltpu.VMEM((1, Cin), jnp.float32),
                        pltpu.VMEM((Cin, Cin), jnp.float32),
                        pltpu.VMEM((Cout, 2), jnp.float32)],
        compiler_params=pltpu.CompilerParams(
            dimension_semantics=("arbitrary",)),
    )(x3, w, w_bf, gb)

    return out3.reshape(N, Cout, Ho, Wo)
